# Initial kernel scaffold; baseline (speedup 1.0000x reference)
import jax
import jax.numpy as jnp
from jax.experimental import pallas as pl


def kernel(x, edge_index, hidden_state, gcn_weight, gcn_bias, w_ih, w_hh, b_ih, b_hh):
    raise NotImplementedError("write your pallas kernel here")


# speedup vs baseline: 15.0160x; 15.0160x over previous
import functools

import jax
import jax.numpy as jnp
from jax import lax
from jax.experimental import pallas as pl
from jax.experimental.pallas import tpu as pltpu
from jax.experimental.pallas import tpu_sc as plsc

NC = 2
NS = 16
CHUNK = 80
DEGW = 16


def _deg_kernel(E, N):
    epc = E // NC
    ept = epc // NS
    nchunk = ept // CHUNK
    npad = -(-N // (8 * NS)) * (8 * NS)
    rpt = npad // NS
    mesh = plsc.VectorSubcoreMesh(core_axis_name="c", subcore_axis_name="s")

    @functools.partial(
        pl.kernel,
        mesh=mesh,
        out_type=jax.ShapeDtypeStruct((NC * npad,), jnp.float32),
        scratch_types=[
            pltpu.VMEM((CHUNK,), jnp.int32),
            pltpu.VMEM((CHUNK,), jnp.float32),
            pltpu.VMEM((rpt,), jnp.float32),
            pltpu.VMEM_SHARED((npad,), jnp.float32),
        ],
    )
    def deg(dst_hbm, zeros_hbm, ones_hbm, out_hbm, dstb, onesb, zbuf, acc):
        c = lax.axis_index("c")
        s = lax.axis_index("s")
        pltpu.sync_copy(ones_hbm, onesb)
        pltpu.sync_copy(zeros_hbm.at[pl.ds(s * rpt, rpt)], zbuf)
        pltpu.sync_copy(zbuf, acc.at[pl.ds(s * rpt, rpt)])
        plsc.subcore_barrier()
        base = c * epc + s * ept

        def body(i, carry):
            pltpu.sync_copy(dst_hbm.at[pl.ds(base + i * CHUNK, CHUNK)], dstb)
            pltpu.sync_copy(onesb, acc.at[dstb], add=True)
            return carry

        lax.fori_loop(0, nchunk, body, 0)
        plsc.subcore_barrier()
        pltpu.sync_copy(acc.at[pl.ds(s * rpt, rpt)], zbuf)
        pltpu.sync_copy(zbuf, out_hbm.at[pl.ds(c * npad + s * rpt, rpt)])

    return deg


def _agg_kernel(E, N, H):
    epc = E // NC
    ept = epc // NS
    nchunk = ept // CHUNK
    npad = -(-N // (8 * NS)) * (8 * NS)
    rpt = npad // NS
    mesh = plsc.VectorSubcoreMesh(core_axis_name="c", subcore_axis_name="s")

    @functools.partial(
        pl.kernel,
        mesh=mesh,
        out_type=jax.ShapeDtypeStruct((NC, npad, H), jnp.float32),
        scratch_types=[
            pltpu.VMEM((CHUNK,), jnp.int32),
            pltpu.VMEM((CHUNK,), jnp.int32),
            pltpu.VMEM((CHUNK, H), jnp.float32),
            pltpu.VMEM_SHARED((npad, H), jnp.float32),
            pltpu.SemaphoreType.DMA,
        ],
    )
    def agg(xws_hbm, src_hbm, dst_hbm, zeros_hbm, out_hbm,
            srcb, dstb, rows, acc, sem):
        c = lax.axis_index("c")
        s = lax.axis_index("s")
        pltpu.sync_copy(zeros_hbm.at[pl.ds(s * rpt, rpt)],
                        acc.at[pl.ds(s * rpt, rpt)])
        plsc.subcore_barrier()
        base = c * epc + s * ept

        def body(i, carry):
            off = base + i * CHUNK
            pltpu.sync_copy(src_hbm.at[pl.ds(off, CHUNK)], srcb)
            pltpu.sync_copy(dst_hbm.at[pl.ds(off, CHUNK)], dstb)
            pltpu.async_copy(xws_hbm.at[srcb], rows, sem).wait()
            pltpu.sync_copy(rows, acc.at[dstb], add=True)
            return carry

        lax.fori_loop(0, nchunk, body, 0)
        plsc.subcore_barrier()
        pltpu.sync_copy(acc.at[pl.ds(s * rpt, rpt)],
                        out_hbm.at[c, pl.ds(s * rpt, rpt)])

    return agg


def _pre_call(x, gcn_weight, cnt, h0, whh_t, bhh, R):
    N, F = x.shape
    H3 = whh_t.shape[1]

    def body(x_ref, w_ref, cnt_ref, h0_ref, whht_ref, bhh_ref,
             xws_ref, gh_ref):
        deg = cnt_ref[0, :, 0:1] + cnt_ref[1, :, 0:1] + 1.0
        dinv = lax.rsqrt(deg)
        xw = jnp.dot(x_ref[...], w_ref[...],
                     preferred_element_type=jnp.float32)
        xws_ref[...] = xw * dinv
        gh_ref[...] = jnp.dot(h0_ref[...], whht_ref[...],
                              preferred_element_type=jnp.float32) + bhh_ref[...]

    grid = (N // R,)
    return pl.pallas_call(
        body,
        grid=grid,
        in_specs=[
            pl.BlockSpec((R, F), lambda i: (i, 0)),
            pl.BlockSpec((F, gcn_weight.shape[1]), lambda i: (0, 0)),
            pl.BlockSpec((NC, R, 1), lambda i: (0, i, 0)),
            pl.BlockSpec((R, H3 // 3), lambda i: (i, 0)),
            pl.BlockSpec((whh_t.shape[0], H3), lambda i: (0, 0)),
            pl.BlockSpec((1, H3), lambda i: (0, 0)),
        ],
        out_specs=[
            pl.BlockSpec((R, gcn_weight.shape[1]), lambda i: (i, 0)),
            pl.BlockSpec((R, H3), lambda i: (i, 0)),
        ],
        out_shape=[
            jax.ShapeDtypeStruct((N, gcn_weight.shape[1]), jnp.float32),
            jax.ShapeDtypeStruct((N, H3), jnp.float32),
        ],
    )(x, gcn_weight, cnt, h0, whh_t, bhh)


def _post_call(aggp, xws, cnt, gcn_bias, gh, wih_t, bih, h0, R):
    N, H = xws.shape
    H3 = 3 * H

    def body(agg_ref, xws_ref, cnt_ref, b_ref, gh_ref, wiht_ref, bih_ref,
             h0_ref, out_ref):
        deg = cnt_ref[0, :, 0:1] + cnt_ref[1, :, 0:1] + 1.0
        dinv = lax.rsqrt(deg)
        xws = xws_ref[...]
        hs = (agg_ref[0] + agg_ref[1] + xws) * dinv + b_ref[...]
        hs = jnp.maximum(hs, 0.0)
        gi = jnp.dot(hs, wiht_ref[...],
                     preferred_element_type=jnp.float32) + bih_ref[...]
        gh = gh_ref[...]
        h0v = h0_ref[...]
        r = jax.nn.sigmoid(gi[:, :H] + gh[:, :H])
        z = jax.nn.sigmoid(gi[:, H:2 * H] + gh[:, H:2 * H])
        n = jnp.tanh(gi[:, 2 * H:] + r * gh[:, 2 * H:])
        out_ref[...] = (1.0 - z) * n + z * h0v

    grid = (N // R,)
    return pl.pallas_call(
        body,
        grid=grid,
        in_specs=[
            pl.BlockSpec((NC, R, H), lambda i: (0, i, 0)),
            pl.BlockSpec((R, H), lambda i: (i, 0)),
            pl.BlockSpec((NC, R, 1), lambda i: (0, i, 0)),
            pl.BlockSpec((1, H), lambda i: (0, 0)),
            pl.BlockSpec((R, H3), lambda i: (i, 0)),
            pl.BlockSpec((H, H3), lambda i: (0, 0)),
            pl.BlockSpec((1, H3), lambda i: (0, 0)),
            pl.BlockSpec((R, H), lambda i: (i, 0)),
        ],
        out_specs=pl.BlockSpec((R, H), lambda i: (i, 0)),
        out_shape=jax.ShapeDtypeStruct((N, H), jnp.float32),
    )(aggp, xws, cnt, gcn_bias, gh, wih_t, bih, h0)


def kernel(x, edge_index, hidden_state, gcn_weight, gcn_bias,
           w_ih, w_hh, b_ih, b_hh):
    N, F = x.shape
    E = edge_index.shape[1]
    H = gcn_weight.shape[1]
    src = edge_index[0]
    dst = edge_index[1]
    h0 = hidden_state[0]
    whh_t = w_hh.T
    wih_t = w_ih.T
    bhh = b_hh.reshape(1, -1)
    bih = b_ih.reshape(1, -1)
    bias = gcn_bias.reshape(1, -1)

    npad = -(-N // (8 * NS)) * (8 * NS)

    zeros_agg = jnp.zeros((npad, H), jnp.float32)
    zeros_deg = jnp.zeros((npad,), jnp.float32)
    ones_deg = jnp.ones((CHUNK,), jnp.float32)
    cnt = _deg_kernel(E, N)(dst, zeros_deg, ones_deg).reshape(NC, npad, 1)
    xws, gh = _pre_call(x, gcn_weight, cnt, h0, whh_t, bhh, R=1000)
    aggp = _agg_kernel(E, N, H)(xws, src, dst, zeros_agg)
    h_new = _post_call(aggp, xws, cnt, bias, gh, wih_t, bih, h0, R=1000)
    return h_new, h_new[None]


# speedup vs baseline: 34.4344x; 2.2932x over previous
import functools

import jax
import jax.numpy as jnp
from jax import lax
from jax.experimental import pallas as pl
from jax.experimental.pallas import tpu as pltpu
from jax.experimental.pallas import tpu_sc as plsc

NC = 2
NS = 16
CK = 128
NB = 2


def _tile_plan(E):
    epc = E // NC
    cks = epc // CK
    cpt = -(-cks // NS)
    cpt = -(-cpt // (2 * NB)) * (2 * NB)
    rows_pad = cpt * NS
    return epc, cks, cpt, rows_pad


def _pad_edges(idx, E):
    _, cks, _, rows_pad = _tile_plan(E)
    r = idx.reshape(NC, cks, CK)
    r = jnp.pad(r, ((0, 0), (0, rows_pad - cks), (0, 0)))
    return r.reshape(NC * rows_pad, CK)


def _deg_kernel(E, N):
    _, cks, cpt, rows_pad = _tile_plan(E)
    npad = -(-N // (8 * NS)) * (8 * NS)
    rpt = npad // NS
    mesh = plsc.VectorSubcoreMesh(core_axis_name="c", subcore_axis_name="s")

    @functools.partial(
        pl.kernel,
        mesh=mesh,
        out_type=jax.ShapeDtypeStruct((NC * npad,), jnp.float32),
        scratch_types=[
            pltpu.VMEM((cpt, CK), jnp.int32),
            pltpu.VMEM((CK,), jnp.float32),
            pltpu.VMEM((rpt,), jnp.float32),
            pltpu.VMEM_SHARED((npad,), jnp.float32),
            pltpu.SemaphoreType.DMA,
        ],
    )
    def deg(dst2_hbm, zeros_hbm, ones_hbm, out_hbm, dstb, onesb, zbuf, acc,
            sem):
        c = lax.axis_index("c")
        s = lax.axis_index("s")
        ct = jnp.minimum(jnp.maximum(cks - s * cpt, 0), cpt)
        pltpu.sync_copy(ones_hbm, onesb)
        pltpu.sync_copy(
            dst2_hbm.at[pl.ds(c * rows_pad + s * cpt, cpt)], dstb)
        pltpu.sync_copy(zeros_hbm.at[pl.ds(s * rpt, rpt)], zbuf)
        pltpu.sync_copy(zbuf, acc.at[pl.ds(s * rpt, rpt)])
        plsc.subcore_barrier()

        lag = 8

        def issue(j, carry):
            @pl.when(j < ct)
            def _():
                pltpu.async_copy(onesb, acc.at[dstb.at[j]], sem, add=True)

            @pl.when(jnp.logical_and(j >= lag, j - lag < ct))
            def _():
                pltpu.make_async_copy(onesb, acc.at[dstb.at[0]], sem).wait()

            return carry

        lax.fori_loop(0, cpt, issue, 0)

        def drain(j, carry):
            @pl.when(j + (cpt - lag) < ct)
            def _():
                pltpu.make_async_copy(onesb, acc.at[dstb.at[0]], sem).wait()

            return carry

        lax.fori_loop(0, lag, drain, 0)
        plsc.subcore_barrier()
        pltpu.sync_copy(acc.at[pl.ds(s * rpt, rpt)], zbuf)
        pltpu.sync_copy(zbuf, out_hbm.at[pl.ds(c * npad + s * rpt, rpt)])

    return deg


def _agg_kernel(E, N, H):
    _, cks, cpt, rows_pad = _tile_plan(E)
    npad = -(-N // (8 * NS)) * (8 * NS)
    rpt = npad // NS
    mesh = plsc.VectorSubcoreMesh(core_axis_name="c", subcore_axis_name="s")
    NI = 2 * NB

    scratch = []
    scratch += [pltpu.VMEM((CK,), jnp.int32) for _ in range(NI)]
    scratch += [pltpu.VMEM((CK,), jnp.int32) for _ in range(NI)]
    scratch += [pltpu.VMEM((CK, H), jnp.float32) for _ in range(NB)]
    scratch += [pltpu.SemaphoreType.DMA for _ in range(NI + 2 * NB)]
    scratch += [pltpu.VMEM_SHARED((npad, H), jnp.float32)]

    @functools.partial(
        pl.kernel,
        mesh=mesh,
        out_type=jax.ShapeDtypeStruct((NC, npad, H), jnp.float32),
        scratch_types=scratch,
    )
    def agg(xws_hbm, src_hbm, dst_hbm, zeros_hbm, out_hbm, *rest):
        srci = rest[:NI]
        dsti = rest[NI:2 * NI]
        rows = rest[2 * NI:2 * NI + NB]
        isem = rest[2 * NI + NB:3 * NI + NB]
        gsem = rest[3 * NI + NB:3 * NI + 2 * NB]
        ssem = rest[3 * NI + 2 * NB:3 * NI + 3 * NB]
        acc = rest[3 * NI + 3 * NB]
        c = lax.axis_index("c")
        s = lax.axis_index("s")
        ct = jnp.minimum(jnp.maximum(cks - s * cpt, 0), cpt)
        base = (c * rows_pad + s * cpt) * CK

        def istart(q, j):
            off = base + j * CK
            pltpu.async_copy(src_hbm.at[pl.ds(off, CK)], srci[q], isem[q])
            pltpu.async_copy(dst_hbm.at[pl.ds(off, CK)], dsti[q], isem[q])

        def iwait(q):
            pltpu.make_async_copy(src_hbm.at[pl.ds(0, CK)], srci[q],
                                  isem[q]).wait()
            pltpu.make_async_copy(dst_hbm.at[pl.ds(0, CK)], dsti[q],
                                  isem[q]).wait()

        def gstart(b, q):
            pltpu.async_copy(xws_hbm.at[srci[q]], rows[b], gsem[b])

        def gwait(b):
            pltpu.make_async_copy(xws_hbm.at[pl.ds(0, CK)], rows[b],
                                  gsem[b]).wait()

        def sstart(b, q):
            pltpu.async_copy(rows[b], acc.at[dsti[q]], ssem[b], add=True)

        def swait(b, q):
            pltpu.make_async_copy(rows[b], acc.at[dsti[q]], ssem[b]).wait()

        pltpu.sync_copy(zeros_hbm.at[pl.ds(s * rpt, rpt)],
                        acc.at[pl.ds(s * rpt, rpt)])
        plsc.subcore_barrier()

        for q in range(NI):
            @pl.when(q < ct)
            def _(q=q):
                istart(q, q)
        for b in range(NB):
            @pl.when(b < ct)
            def _(b=b):
                iwait(b)
                gstart(b, b)

        def body(k, carry):
            for u in range(NI):
                j = k * NI + u
                b = u % NB
                q = u

                @pl.when(j < ct)
                def _(j=j, b=b, q=q):
                    gwait(b)
                    sstart(b, q)

                @pl.when(j + NB < ct)
                def _(j=j, b=b, q=q):
                    swait(b, q)
                    iwait((q + NB) % NI)
                    gstart(b, (q + NB) % NI)

                @pl.when(jnp.logical_and(j + NB >= ct, j < ct))
                def _(j=j, b=b, q=q):
                    swait(b, q)

                @pl.when(j + NI < ct)
                def _(j=j, q=q):
                    istart(q, j + NI)


            return carry

        lax.fori_loop(0, cpt // NI, body, 0)
        plsc.subcore_barrier()
        pltpu.sync_copy(acc.at[pl.ds(s * rpt, rpt)],
                        out_hbm.at[c, pl.ds(s * rpt, rpt)])

    return agg


def _pre_call(x, gcn_weight, cnt, h0, whh_t, bhh, R):
    N, F = x.shape
    H3 = whh_t.shape[1]

    def body(x_ref, w_ref, cnt_ref, h0_ref, whht_ref, bhh_ref,
             xws_ref, gh_ref):
        deg = cnt_ref[0, :, 0:1] + cnt_ref[1, :, 0:1] + 1.0
        dinv = lax.rsqrt(deg)
        xw = jnp.dot(x_ref[...], w_ref[...],
                     preferred_element_type=jnp.float32)
        xws_ref[...] = xw * dinv
        gh_ref[...] = jnp.dot(h0_ref[...], whht_ref[...],
                              preferred_element_type=jnp.float32) + bhh_ref[...]

    grid = (N // R,)
    return pl.pallas_call(
        body,
        grid=grid,
        in_specs=[
            pl.BlockSpec((R, F), lambda i: (i, 0)),
            pl.BlockSpec((F, gcn_weight.shape[1]), lambda i: (0, 0)),
            pl.BlockSpec((NC, R, 1), lambda i: (0, i, 0)),
            pl.BlockSpec((R, H3 // 3), lambda i: (i, 0)),
            pl.BlockSpec((whh_t.shape[0], H3), lambda i: (0, 0)),
            pl.BlockSpec((1, H3), lambda i: (0, 0)),
        ],
        out_specs=[
            pl.BlockSpec((R, gcn_weight.shape[1]), lambda i: (i, 0)),
            pl.BlockSpec((R, H3), lambda i: (i, 0)),
        ],
        out_shape=[
            jax.ShapeDtypeStruct((N, gcn_weight.shape[1]), jnp.float32),
            jax.ShapeDtypeStruct((N, H3), jnp.float32),
        ],
    )(x, gcn_weight, cnt, h0, whh_t, bhh)


def _post_call(aggp, xws, cnt, gcn_bias, gh, wih_t, bih, h0, R):
    N, H = xws.shape
    H3 = 3 * H

    def body(agg_ref, xws_ref, cnt_ref, b_ref, gh_ref, wiht_ref, bih_ref,
             h0_ref, out_ref):
        deg = cnt_ref[0, :, 0:1] + cnt_ref[1, :, 0:1] + 1.0
        dinv = lax.rsqrt(deg)
        xws = xws_ref[...]
        hs = (agg_ref[0] + agg_ref[1] + xws) * dinv + b_ref[...]
        hs = jnp.maximum(hs, 0.0)
        gi = jnp.dot(hs, wiht_ref[...],
                     preferred_element_type=jnp.float32) + bih_ref[...]
        gh = gh_ref[...]
        h0v = h0_ref[...]
        r = jax.nn.sigmoid(gi[:, :H] + gh[:, :H])
        z = jax.nn.sigmoid(gi[:, H:2 * H] + gh[:, H:2 * H])
        n = jnp.tanh(gi[:, 2 * H:] + r * gh[:, 2 * H:])
        out_ref[...] = (1.0 - z) * n + z * h0v

    grid = (N // R,)
    return pl.pallas_call(
        body,
        grid=grid,
        in_specs=[
            pl.BlockSpec((NC, R, H), lambda i: (0, i, 0)),
            pl.BlockSpec((R, H), lambda i: (i, 0)),
            pl.BlockSpec((NC, R, 1), lambda i: (0, i, 0)),
            pl.BlockSpec((1, H), lambda i: (0, 0)),
            pl.BlockSpec((R, H3), lambda i: (i, 0)),
            pl.BlockSpec((H, H3), lambda i: (0, 0)),
            pl.BlockSpec((1, H3), lambda i: (0, 0)),
            pl.BlockSpec((R, H), lambda i: (i, 0)),
        ],
        out_specs=pl.BlockSpec((R, H), lambda i: (i, 0)),
        out_shape=jax.ShapeDtypeStruct((N, H), jnp.float32),
    )(aggp, xws, cnt, gcn_bias, gh, wih_t, bih, h0)


def kernel(x, edge_index, hidden_state, gcn_weight, gcn_bias,
           w_ih, w_hh, b_ih, b_hh):
    N, F = x.shape
    E = edge_index.shape[1]
    H = gcn_weight.shape[1]
    dst2 = _pad_edges(edge_index[1], E)
    srcf = _pad_edges(edge_index[0], E).reshape(-1)
    dstf = dst2.reshape(-1)
    h0 = hidden_state[0]
    whh_t = w_hh.T
    wih_t = w_ih.T
    bhh = b_hh.reshape(1, -1)
    bih = b_ih.reshape(1, -1)
    bias = gcn_bias.reshape(1, -1)

    npad = -(-N // (8 * NS)) * (8 * NS)
    zeros_agg = jnp.zeros((npad, H), jnp.float32)
    zeros_deg = jnp.zeros((npad,), jnp.float32)
    ones_deg = jnp.ones((CK,), jnp.float32)

    cnt = _deg_kernel(E, N)(dst2, zeros_deg, ones_deg).reshape(NC, npad, 1)
    xws, gh = _pre_call(x, gcn_weight, cnt, h0, whh_t, bhh, R=1000)
    aggp = _agg_kernel(E, N, H)(xws, srcf, dstf, zeros_agg)
    h_new = _post_call(aggp, xws, cnt, bias, gh, wih_t, bih, h0, R=1000)
    return h_new, h_new[None]


# speedup vs baseline: 36.3149x; 1.0546x over previous
import functools

import jax
import jax.numpy as jnp
from jax import lax
from jax.experimental import pallas as pl
from jax.experimental.pallas import tpu as pltpu
from jax.experimental.pallas import tpu_sc as plsc

NC = 2
NS = 16
CK = 64
NB = 4


def _tile_plan(E):
    epc = E // NC
    cks = epc // CK
    cpt = -(-cks // NS)
    cpt = -(-cpt // (2 * NB)) * (2 * NB)
    rows_pad = cpt * NS
    return epc, cks, cpt, rows_pad


def _pad_edges(idx, E):
    _, cks, _, rows_pad = _tile_plan(E)
    r = idx.reshape(NC, cks, CK)
    r = jnp.pad(r, ((0, 0), (0, rows_pad - cks), (0, 0)))
    return r.reshape(NC * rows_pad, CK)


def _deg_kernel(E, N):
    _, cks, cpt, rows_pad = _tile_plan(E)
    npad = -(-N // (8 * NS)) * (8 * NS)
    rpt = npad // NS
    mesh = plsc.VectorSubcoreMesh(core_axis_name="c", subcore_axis_name="s")

    @functools.partial(
        pl.kernel,
        mesh=mesh,
        out_type=jax.ShapeDtypeStruct((NC * npad,), jnp.float32),
        scratch_types=[
            pltpu.VMEM((cpt, CK), jnp.int32),
            pltpu.VMEM((CK,), jnp.float32),
            pltpu.VMEM((rpt,), jnp.float32),
            pltpu.VMEM_SHARED((npad,), jnp.float32),
            pltpu.SemaphoreType.DMA,
        ],
    )
    def deg(dst2_hbm, zeros_hbm, ones_hbm, out_hbm, dstb, onesb, zbuf, acc,
            sem):
        c = lax.axis_index("c")
        s = lax.axis_index("s")
        ct = jnp.minimum(jnp.maximum(cks - s * cpt, 0), cpt)
        pltpu.sync_copy(ones_hbm, onesb)
        pltpu.sync_copy(
            dst2_hbm.at[pl.ds(c * rows_pad + s * cpt, cpt)], dstb)
        pltpu.sync_copy(zeros_hbm.at[pl.ds(s * rpt, rpt)], zbuf)
        pltpu.sync_copy(zbuf, acc.at[pl.ds(s * rpt, rpt)])
        plsc.subcore_barrier()

        lag = 8

        def issue(j, carry):
            @pl.when(j < ct)
            def _():
                pltpu.async_copy(onesb, acc.at[dstb.at[j]], sem, add=True)

            @pl.when(jnp.logical_and(j >= lag, j - lag < ct))
            def _():
                pltpu.make_async_copy(onesb, acc.at[dstb.at[0]], sem).wait()

            return carry

        lax.fori_loop(0, cpt, issue, 0)

        def drain(j, carry):
            @pl.when(j + (cpt - lag) < ct)
            def _():
                pltpu.make_async_copy(onesb, acc.at[dstb.at[0]], sem).wait()

            return carry

        lax.fori_loop(0, lag, drain, 0)
        plsc.subcore_barrier()
        pltpu.sync_copy(acc.at[pl.ds(s * rpt, rpt)], zbuf)
        pltpu.sync_copy(zbuf, out_hbm.at[pl.ds(c * npad + s * rpt, rpt)])

    return deg


def _agg_kernel(E, N, H):
    _, cks, cpt, rows_pad = _tile_plan(E)
    npad = -(-N // (8 * NS)) * (8 * NS)
    rpt = npad // NS
    mesh = plsc.VectorSubcoreMesh(core_axis_name="c", subcore_axis_name="s")
    NI = 2 * NB

    scratch = []
    scratch += [pltpu.VMEM((CK,), jnp.int32) for _ in range(NI)]
    scratch += [pltpu.VMEM((CK,), jnp.int32) for _ in range(NI)]
    scratch += [pltpu.VMEM((CK, H), jnp.float32) for _ in range(NB)]
    scratch += [pltpu.SemaphoreType.DMA for _ in range(NI + 2 * NB)]
    scratch += [pltpu.VMEM_SHARED((npad, H), jnp.float32)]

    @functools.partial(
        pl.kernel,
        mesh=mesh,
        out_type=jax.ShapeDtypeStruct((NC, npad, H), jnp.float32),
        scratch_types=scratch,
    )
    def agg(xws_hbm, src_hbm, dst_hbm, zeros_hbm, out_hbm, *rest):
        srci = rest[:NI]
        dsti = rest[NI:2 * NI]
        rows = rest[2 * NI:2 * NI + NB]
        isem = rest[2 * NI + NB:3 * NI + NB]
        gsem = rest[3 * NI + NB:3 * NI + 2 * NB]
        ssem = rest[3 * NI + 2 * NB:3 * NI + 3 * NB]
        acc = rest[3 * NI + 3 * NB]
        c = lax.axis_index("c")
        s = lax.axis_index("s")
        ct = jnp.minimum(jnp.maximum(cks - s * cpt, 0), cpt)
        base = (c * rows_pad + s * cpt) * CK

        def istart(q, j):
            off = base + j * CK
            pltpu.async_copy(src_hbm.at[pl.ds(off, CK)], srci[q], isem[q])
            pltpu.async_copy(dst_hbm.at[pl.ds(off, CK)], dsti[q], isem[q])

        def iwait(q):
            pltpu.make_async_copy(src_hbm.at[pl.ds(0, CK)], srci[q],
                                  isem[q]).wait()
            pltpu.make_async_copy(dst_hbm.at[pl.ds(0, CK)], dsti[q],
                                  isem[q]).wait()

        def gstart(b, q):
            pltpu.async_copy(xws_hbm.at[srci[q]], rows[b], gsem[b])

        def gwait(b):
            pltpu.make_async_copy(xws_hbm.at[pl.ds(0, CK)], rows[b],
                                  gsem[b]).wait()

        def sstart(b, q):
            pltpu.async_copy(rows[b], acc.at[dsti[q]], ssem[b], add=True)

        def swait(b, q):
            pltpu.make_async_copy(rows[b], acc.at[dsti[q]], ssem[b]).wait()

        pltpu.sync_copy(zeros_hbm.at[pl.ds(s * rpt, rpt)],
                        acc.at[pl.ds(s * rpt, rpt)])
        plsc.subcore_barrier()

        for q in range(NI):
            @pl.when(q < ct)
            def _(q=q):
                istart(q, q)
        for b in range(NB):
            @pl.when(b < ct)
            def _(b=b):
                iwait(b)
                gstart(b, b)

        def body(k, carry):
            for u in range(NI):
                j = k * NI + u
                b = u % NB
                q = u

                @pl.when(j < ct)
                def _(j=j, b=b, q=q):
                    gwait(b)
                    sstart(b, q)

                @pl.when(j + NB < ct)
                def _(j=j, b=b, q=q):
                    swait(b, q)
                    iwait((q + NB) % NI)
                    gstart(b, (q + NB) % NI)

                @pl.when(jnp.logical_and(j + NB >= ct, j < ct))
                def _(j=j, b=b, q=q):
                    swait(b, q)

                @pl.when(j + NI < ct)
                def _(j=j, q=q):
                    istart(q, j + NI)


            return carry

        lax.fori_loop(0, cpt // NI, body, 0)
        plsc.subcore_barrier()
        pltpu.sync_copy(acc.at[pl.ds(s * rpt, rpt)],
                        out_hbm.at[c, pl.ds(s * rpt, rpt)])

    return agg


def _pre_call(x, gcn_weight, cnt, h0, whh_t, bhh, R):
    N, F = x.shape
    H3 = whh_t.shape[1]

    def body(x_ref, w_ref, cnt_ref, h0_ref, whht_ref, bhh_ref,
             xws_ref, gh_ref):
        deg = cnt_ref[0, :, 0:1] + cnt_ref[1, :, 0:1] + 1.0
        dinv = lax.rsqrt(deg)
        xw = jnp.dot(x_ref[...], w_ref[...],
                     preferred_element_type=jnp.float32)
        xws_ref[...] = xw * dinv
        gh_ref[...] = jnp.dot(h0_ref[...], whht_ref[...],
                              preferred_element_type=jnp.float32) + bhh_ref[...]

    grid = (N // R,)
    return pl.pallas_call(
        body,
        grid=grid,
        in_specs=[
            pl.BlockSpec((R, F), lambda i: (i, 0)),
            pl.BlockSpec((F, gcn_weight.shape[1]), lambda i: (0, 0)),
            pl.BlockSpec((NC, R, 1), lambda i: (0, i, 0)),
            pl.BlockSpec((R, H3 // 3), lambda i: (i, 0)),
            pl.BlockSpec((whh_t.shape[0], H3), lambda i: (0, 0)),
            pl.BlockSpec((1, H3), lambda i: (0, 0)),
        ],
        out_specs=[
            pl.BlockSpec((R, gcn_weight.shape[1]), lambda i: (i, 0)),
            pl.BlockSpec((R, H3), lambda i: (i, 0)),
        ],
        out_shape=[
            jax.ShapeDtypeStruct((N, gcn_weight.shape[1]), jnp.float32),
            jax.ShapeDtypeStruct((N, H3), jnp.float32),
        ],
    )(x, gcn_weight, cnt, h0, whh_t, bhh)


def _post_call(aggp, xws, cnt, gcn_bias, gh, wih_t, bih, h0, R):
    N, H = xws.shape
    H3 = 3 * H

    def body(agg_ref, xws_ref, cnt_ref, b_ref, gh_ref, wiht_ref, bih_ref,
             h0_ref, out_ref):
        deg = cnt_ref[0, :, 0:1] + cnt_ref[1, :, 0:1] + 1.0
        dinv = lax.rsqrt(deg)
        xws = xws_ref[...]
        hs = (agg_ref[0] + agg_ref[1] + xws) * dinv + b_ref[...]
        hs = jnp.maximum(hs, 0.0)
        gi = jnp.dot(hs, wiht_ref[...],
                     preferred_element_type=jnp.float32) + bih_ref[...]
        gh = gh_ref[...]
        h0v = h0_ref[...]
        r = jax.nn.sigmoid(gi[:, :H] + gh[:, :H])
        z = jax.nn.sigmoid(gi[:, H:2 * H] + gh[:, H:2 * H])
        n = jnp.tanh(gi[:, 2 * H:] + r * gh[:, 2 * H:])
        out_ref[...] = (1.0 - z) * n + z * h0v

    grid = (N // R,)
    return pl.pallas_call(
        body,
        grid=grid,
        in_specs=[
            pl.BlockSpec((NC, R, H), lambda i: (0, i, 0)),
            pl.BlockSpec((R, H), lambda i: (i, 0)),
            pl.BlockSpec((NC, R, 1), lambda i: (0, i, 0)),
            pl.BlockSpec((1, H), lambda i: (0, 0)),
            pl.BlockSpec((R, H3), lambda i: (i, 0)),
            pl.BlockSpec((H, H3), lambda i: (0, 0)),
            pl.BlockSpec((1, H3), lambda i: (0, 0)),
            pl.BlockSpec((R, H), lambda i: (i, 0)),
        ],
        out_specs=pl.BlockSpec((R, H), lambda i: (i, 0)),
        out_shape=jax.ShapeDtypeStruct((N, H), jnp.float32),
    )(aggp, xws, cnt, gcn_bias, gh, wih_t, bih, h0)


def kernel(x, edge_index, hidden_state, gcn_weight, gcn_bias,
           w_ih, w_hh, b_ih, b_hh):
    N, F = x.shape
    E = edge_index.shape[1]
    H = gcn_weight.shape[1]
    dst2 = _pad_edges(edge_index[1], E)
    srcf = _pad_edges(edge_index[0], E).reshape(-1)
    dstf = dst2.reshape(-1)
    h0 = hidden_state[0]
    whh_t = w_hh.T
    wih_t = w_ih.T
    bhh = b_hh.reshape(1, -1)
    bih = b_ih.reshape(1, -1)
    bias = gcn_bias.reshape(1, -1)

    npad = -(-N // (8 * NS)) * (8 * NS)
    zeros_agg = jnp.zeros((npad, H), jnp.float32)
    zeros_deg = jnp.zeros((npad,), jnp.float32)
    ones_deg = jnp.ones((CK,), jnp.float32)

    cnt = _deg_kernel(E, N)(dst2, zeros_deg, ones_deg).reshape(NC, npad, 1)
    xws, gh = _pre_call(x, gcn_weight, cnt, h0, whh_t, bhh, R=1000)
    aggp = _agg_kernel(E, N, H)(xws, srcf, dstf, zeros_agg)
    h_new = _post_call(aggp, xws, cnt, bias, gh, wih_t, bih, h0, R=1000)
    return h_new, h_new[None]


# speedup vs baseline: 37.4210x; 1.0305x over previous
import functools

import jax
import jax.numpy as jnp
from jax import lax
from jax.experimental import pallas as pl
from jax.experimental.pallas import tpu as pltpu
from jax.experimental.pallas import tpu_sc as plsc

NC = 2
NS = 16
CK = 64
NB = 4


def _tile_plan(E):
    epc = E // NC
    cks = epc // CK
    cpt = -(-cks // NS)
    cpt = -(-cpt // (2 * NB)) * (2 * NB)
    rows_pad = cpt * NS
    return epc, cks, cpt, rows_pad


def _pad_edges(idx, E):
    _, cks, _, rows_pad = _tile_plan(E)
    r = idx.reshape(NC, cks, CK)
    r = jnp.pad(r, ((0, 0), (0, rows_pad - cks), (0, 0)))
    return r.reshape(NC * rows_pad, CK)


def _deg_kernel(E, N):
    _, cks, cpt, rows_pad = _tile_plan(E)
    npad = -(-N // (8 * NS)) * (8 * NS)
    rpt = npad // NS
    mesh = plsc.VectorSubcoreMesh(core_axis_name="c", subcore_axis_name="s")
    NI = 8
    lag = 4

    scratch = [pltpu.VMEM((CK,), jnp.int32) for _ in range(NI)]
    scratch += [
        pltpu.VMEM((CK,), jnp.float32),
        pltpu.VMEM((rpt,), jnp.float32),
        pltpu.VMEM_SHARED((npad,), jnp.float32),
    ]
    scratch += [pltpu.SemaphoreType.DMA for _ in range(NI + 1)]

    @functools.partial(
        pl.kernel,
        mesh=mesh,
        out_type=jax.ShapeDtypeStruct((NC * npad,), jnp.float32),
        scratch_types=scratch,
    )
    def deg(dst_hbm, zeros_hbm, ones_hbm, out_hbm, *rest):
        dsti = rest[:NI]
        onesb = rest[NI]
        zbuf = rest[NI + 1]
        acc = rest[NI + 2]
        isem = rest[NI + 3:2 * NI + 3]
        ssem = rest[2 * NI + 3]
        c = lax.axis_index("c")
        s = lax.axis_index("s")
        ct = jnp.minimum(jnp.maximum(cks - s * cpt, 0), cpt)
        base = (c * cks + s * cpt) * CK

        def istart(q, j):
            pltpu.async_copy(dst_hbm.at[pl.ds(base + j * CK, CK)],
                             dsti[q], isem[q])

        def iwait(q):
            pltpu.make_async_copy(dst_hbm.at[pl.ds(0, CK)], dsti[q],
                                  isem[q]).wait()

        pltpu.sync_copy(ones_hbm, onesb)
        pltpu.sync_copy(zeros_hbm, zbuf)
        pltpu.sync_copy(zbuf, acc.at[pl.ds(s * rpt, rpt)])
        plsc.subcore_barrier()

        for q in range(NI):
            @pl.when(q < ct)
            def _(q=q):
                istart(q, q)

        def body(k, carry):
            for q in range(NI):
                j = k * NI + q

                @pl.when(j < ct)
                def _(j=j, q=q):
                    iwait(q)
                    pltpu.async_copy(onesb, acc.at[dsti[q]], ssem, add=True)

                @pl.when(jnp.logical_and(j >= lag, j - lag < ct))
                def _():
                    pltpu.make_async_copy(onesb, acc.at[dsti[0]],
                                          ssem).wait()

                @pl.when(j + NI < ct)
                def _(j=j, q=q):
                    istart(q, j + NI)

            return carry

        lax.fori_loop(0, cpt // NI, body, 0)

        def drain(j, carry):
            @pl.when(j + (cpt - lag) < ct)
            def _():
                pltpu.make_async_copy(onesb, acc.at[dsti[0]], ssem).wait()

            return carry

        lax.fori_loop(0, lag, drain, 0)
        plsc.subcore_barrier()
        pltpu.sync_copy(acc.at[pl.ds(s * rpt, rpt)], zbuf)
        pltpu.sync_copy(zbuf, out_hbm.at[pl.ds(c * npad + s * rpt, rpt)])

    return deg


def _agg_kernel(E, N, H):
    _, cks, cpt, rows_pad = _tile_plan(E)
    npad = -(-N // (8 * NS)) * (8 * NS)
    rpt = npad // NS
    mesh = plsc.VectorSubcoreMesh(core_axis_name="c", subcore_axis_name="s")
    NI = 2 * NB

    scratch = []
    scratch += [pltpu.VMEM((CK,), jnp.int32) for _ in range(NI)]
    scratch += [pltpu.VMEM((CK,), jnp.int32) for _ in range(NI)]
    scratch += [pltpu.VMEM((CK, H), jnp.float32) for _ in range(NB)]
    scratch += [pltpu.SemaphoreType.DMA for _ in range(NI + 2 * NB)]
    scratch += [pltpu.VMEM_SHARED((npad, H), jnp.float32)]

    @functools.partial(
        pl.kernel,
        mesh=mesh,
        out_type=jax.ShapeDtypeStruct((NC, npad, H), jnp.float32),
        scratch_types=scratch,
    )
    def agg(xws_hbm, src_hbm, dst_hbm, zeros_hbm, out_hbm, *rest):
        srci = rest[:NI]
        dsti = rest[NI:2 * NI]
        rows = rest[2 * NI:2 * NI + NB]
        isem = rest[2 * NI + NB:3 * NI + NB]
        gsem = rest[3 * NI + NB:3 * NI + 2 * NB]
        ssem = rest[3 * NI + 2 * NB:3 * NI + 3 * NB]
        acc = rest[3 * NI + 3 * NB]
        c = lax.axis_index("c")
        s = lax.axis_index("s")
        ct = jnp.minimum(jnp.maximum(cks - s * cpt, 0), cpt)
        base = (c * cks + s * cpt) * CK

        def istart(q, j):
            off = base + j * CK
            pltpu.async_copy(src_hbm.at[pl.ds(off, CK)], srci[q], isem[q])
            pltpu.async_copy(dst_hbm.at[pl.ds(off, CK)], dsti[q], isem[q])

        def iwait(q):
            pltpu.make_async_copy(src_hbm.at[pl.ds(0, CK)], srci[q],
                                  isem[q]).wait()
            pltpu.make_async_copy(dst_hbm.at[pl.ds(0, CK)], dsti[q],
                                  isem[q]).wait()

        def gstart(b, q):
            pltpu.async_copy(xws_hbm.at[srci[q]], rows[b], gsem[b])

        def gwait(b):
            pltpu.make_async_copy(xws_hbm.at[pl.ds(0, CK)], rows[b],
                                  gsem[b]).wait()

        def sstart(b, q):
            pltpu.async_copy(rows[b], acc.at[dsti[q]], ssem[b], add=True)

        def swait(b, q):
            pltpu.make_async_copy(rows[b], acc.at[dsti[q]], ssem[b]).wait()

        pltpu.sync_copy(zeros_hbm, acc.at[pl.ds(s * rpt, rpt)])
        plsc.subcore_barrier()

        for q in range(NI):
            @pl.when(q < ct)
            def _(q=q):
                istart(q, q)
        for b in range(NB):
            @pl.when(b < ct)
            def _(b=b):
                iwait(b)
                gstart(b, b)

        def body(k, carry):
            for u in range(NI):
                j = k * NI + u
                b = u % NB
                q = u

                @pl.when(j < ct)
                def _(j=j, b=b, q=q):
                    gwait(b)
                    sstart(b, q)

                @pl.when(j + NB < ct)
                def _(j=j, b=b, q=q):
                    swait(b, q)
                    iwait((q + NB) % NI)
                    gstart(b, (q + NB) % NI)

                @pl.when(jnp.logical_and(j + NB >= ct, j < ct))
                def _(j=j, b=b, q=q):
                    swait(b, q)

                @pl.when(j + NI < ct)
                def _(j=j, q=q):
                    istart(q, j + NI)


            return carry

        lax.fori_loop(0, cpt // NI, body, 0)
        plsc.subcore_barrier()
        pltpu.sync_copy(acc.at[pl.ds(s * rpt, rpt)],
                        out_hbm.at[c, pl.ds(s * rpt, rpt)])

    return agg


def _pre_call(x, gcn_weight, cnt, R):
    N, F = x.shape
    H = gcn_weight.shape[1]

    def body(x_ref, w_ref, cnt_ref, xws_ref):
        deg = cnt_ref[0, :, 0:1] + cnt_ref[1, :, 0:1] + 1.0
        dinv = lax.rsqrt(deg)
        xw = jnp.dot(x_ref[...], w_ref[...],
                     preferred_element_type=jnp.float32)
        xws_ref[...] = xw * dinv

    grid = (N // R,)
    return pl.pallas_call(
        body,
        grid=grid,
        in_specs=[
            pl.BlockSpec((R, F), lambda i: (i, 0)),
            pl.BlockSpec((F, H), lambda i: (0, 0)),
            pl.BlockSpec((NC, R, 1), lambda i: (0, i, 0)),
        ],
        out_specs=pl.BlockSpec((R, H), lambda i: (i, 0)),
        out_shape=jax.ShapeDtypeStruct((N, H), jnp.float32),
    )(x, gcn_weight, cnt)


def _post_call(aggp, xws, cnt, gcn_bias, whh_t, bhh, wih_t, bih, h0, R):
    N, H = xws.shape
    H3 = 3 * H

    def body(agg_ref, xws_ref, cnt_ref, b_ref, whht_ref, bhh_ref,
             wiht_ref, bih_ref, h0_ref, out_ref):
        deg = cnt_ref[0, :, 0:1] + cnt_ref[1, :, 0:1] + 1.0
        dinv = lax.rsqrt(deg)
        xws = xws_ref[...]
        hs = (agg_ref[0] + agg_ref[1] + xws) * dinv + b_ref[...]
        hs = jnp.maximum(hs, 0.0)
        h0v = h0_ref[...]
        gi = jnp.dot(hs, wiht_ref[...],
                     preferred_element_type=jnp.float32) + bih_ref[...]
        gh = jnp.dot(h0v, whht_ref[...],
                     preferred_element_type=jnp.float32) + bhh_ref[...]
        r = jax.nn.sigmoid(gi[:, :H] + gh[:, :H])
        z = jax.nn.sigmoid(gi[:, H:2 * H] + gh[:, H:2 * H])
        n = jnp.tanh(gi[:, 2 * H:] + r * gh[:, 2 * H:])
        out_ref[...] = (1.0 - z) * n + z * h0v

    grid = (N // R,)
    return pl.pallas_call(
        body,
        grid=grid,
        in_specs=[
            pl.BlockSpec((NC, R, H), lambda i: (0, i, 0)),
            pl.BlockSpec((R, H), lambda i: (i, 0)),
            pl.BlockSpec((NC, R, 1), lambda i: (0, i, 0)),
            pl.BlockSpec((1, H), lambda i: (0, 0)),
            pl.BlockSpec((H, H3), lambda i: (0, 0)),
            pl.BlockSpec((1, H3), lambda i: (0, 0)),
            pl.BlockSpec((H, H3), lambda i: (0, 0)),
            pl.BlockSpec((1, H3), lambda i: (0, 0)),
            pl.BlockSpec((R, H), lambda i: (i, 0)),
        ],
        out_specs=pl.BlockSpec((R, H), lambda i: (i, 0)),
        out_shape=jax.ShapeDtypeStruct((N, H), jnp.float32),
    )(aggp, xws, cnt, gcn_bias, whh_t, bhh, wih_t, bih, h0)


def kernel(x, edge_index, hidden_state, gcn_weight, gcn_bias,
           w_ih, w_hh, b_ih, b_hh):
    N, F = x.shape
    E = edge_index.shape[1]
    H = gcn_weight.shape[1]
    srcf = edge_index[0]
    dstf = edge_index[1]
    h0 = hidden_state[0]
    whh_t = w_hh.T
    wih_t = w_ih.T
    bhh = b_hh.reshape(1, -1)
    bih = b_ih.reshape(1, -1)
    bias = gcn_bias.reshape(1, -1)

    npad = -(-N // (8 * NS)) * (8 * NS)
    rpt = npad // NS
    zeros_agg = jnp.zeros((rpt, H), jnp.float32)
    zeros_deg = jnp.zeros((rpt,), jnp.float32)
    ones_deg = jnp.ones((CK,), jnp.float32)

    cnt = _deg_kernel(E, N)(dstf, zeros_deg, ones_deg).reshape(NC, npad, 1)
    xws = _pre_call(x, gcn_weight, cnt, R=1000)
    aggp = _agg_kernel(E, N, H)(xws, srcf, dstf, zeros_agg)
    h_new = _post_call(aggp, xws, cnt, bias, whh_t, bhh, wih_t, bih, h0,
                       R=1000)
    return h_new, h_new[None]


# speedup vs baseline: 40.0728x; 1.0709x over previous
import functools

import jax
import jax.numpy as jnp
from jax import lax
from jax.experimental import pallas as pl
from jax.experimental.pallas import tpu as pltpu
from jax.experimental.pallas import tpu_sc as plsc

NC = 2
NS = 16
CK = 64
NB = 4


def _tile_plan(E):
    epc = E // NC
    cks = epc // CK
    cpt = -(-cks // NS)
    cpt = -(-cpt // (2 * NB)) * (2 * NB)
    rows_pad = cpt * NS
    return epc, cks, cpt, rows_pad


def _pad_edges(idx, E):
    _, cks, _, rows_pad = _tile_plan(E)
    r = idx.reshape(NC, cks, CK)
    r = jnp.pad(r, ((0, 0), (0, rows_pad - cks), (0, 0)))
    return r.reshape(NC * rows_pad, CK)


def _deg_kernel(E, N):
    _, cks, cpt, rows_pad = _tile_plan(E)
    npad = -(-N // (8 * NS)) * (8 * NS)
    rpt = npad // NS
    mesh = plsc.VectorSubcoreMesh(core_axis_name="c", subcore_axis_name="s")
    NI = 8
    lag = 4

    scratch = [pltpu.VMEM((CK,), jnp.int32) for _ in range(NI)]
    scratch += [
        pltpu.VMEM((CK,), jnp.float32),
        pltpu.VMEM((rpt,), jnp.float32),
        pltpu.VMEM_SHARED((npad,), jnp.float32),
    ]
    scratch += [pltpu.SemaphoreType.DMA for _ in range(NI + 1)]

    @functools.partial(
        pl.kernel,
        mesh=mesh,
        out_type=jax.ShapeDtypeStruct((NC * npad,), jnp.float32),
        scratch_types=scratch,
    )
    def deg(ei_hbm, zeros_hbm, ones_hbm, out_hbm, *rest):
        dsti = rest[:NI]
        onesb = rest[NI]
        zbuf = rest[NI + 1]
        acc = rest[NI + 2]
        isem = rest[NI + 3:2 * NI + 3]
        ssem = rest[2 * NI + 3]
        c = lax.axis_index("c")
        s = lax.axis_index("s")
        ct = jnp.minimum(jnp.maximum(cks - s * cpt, 0), cpt)
        base = E + (c * cks + s * cpt) * CK

        def istart(q, j):
            pltpu.async_copy(ei_hbm.at[pl.ds(base + j * CK, CK)],
                             dsti[q], isem[q])

        def iwait(q):
            pltpu.make_async_copy(ei_hbm.at[pl.ds(0, CK)], dsti[q],
                                  isem[q]).wait()

        pltpu.sync_copy(ones_hbm, onesb)
        pltpu.sync_copy(zeros_hbm, zbuf)
        pltpu.sync_copy(zbuf, acc.at[pl.ds(s * rpt, rpt)])
        plsc.subcore_barrier()

        for q in range(NI):
            @pl.when(q < ct)
            def _(q=q):
                istart(q, q)

        def body(k, carry):
            for q in range(NI):
                j = k * NI + q

                @pl.when(j < ct)
                def _(j=j, q=q):
                    iwait(q)
                    pltpu.async_copy(onesb, acc.at[dsti[q]], ssem, add=True)

                @pl.when(jnp.logical_and(j >= lag, j - lag < ct))
                def _():
                    pltpu.make_async_copy(onesb, acc.at[dsti[0]],
                                          ssem).wait()

                @pl.when(j + NI < ct)
                def _(j=j, q=q):
                    istart(q, j + NI)

            return carry

        lax.fori_loop(0, cpt // NI, body, 0)

        def drain(j, carry):
            @pl.when(j + (cpt - lag) < ct)
            def _():
                pltpu.make_async_copy(onesb, acc.at[dsti[0]], ssem).wait()

            return carry

        lax.fori_loop(0, lag, drain, 0)
        plsc.subcore_barrier()
        pltpu.sync_copy(acc.at[pl.ds(s * rpt, rpt)], zbuf)
        pltpu.sync_copy(zbuf, out_hbm.at[pl.ds(c * npad + s * rpt, rpt)])

    return deg


def _agg_kernel(E, N, H):
    _, cks, cpt, rows_pad = _tile_plan(E)
    npad = -(-N // (8 * NS)) * (8 * NS)
    rpt = npad // NS
    mesh = plsc.VectorSubcoreMesh(core_axis_name="c", subcore_axis_name="s")
    NI = 2 * NB

    scratch = []
    scratch += [pltpu.VMEM((CK,), jnp.int32) for _ in range(NI)]
    scratch += [pltpu.VMEM((CK,), jnp.int32) for _ in range(NI)]
    scratch += [pltpu.VMEM((CK, H), jnp.float32) for _ in range(NB)]
    scratch += [pltpu.SemaphoreType.DMA for _ in range(NI + 2 * NB)]
    scratch += [pltpu.VMEM_SHARED((npad, H), jnp.float32)]

    @functools.partial(
        pl.kernel,
        mesh=mesh,
        out_type=jax.ShapeDtypeStruct((NC, npad, H), jnp.float32),
        scratch_types=scratch,
    )
    def agg(xws_hbm, ei_hbm, zeros_hbm, out_hbm, *rest):
        srci = rest[:NI]
        dsti = rest[NI:2 * NI]
        rows = rest[2 * NI:2 * NI + NB]
        isem = rest[2 * NI + NB:3 * NI + NB]
        gsem = rest[3 * NI + NB:3 * NI + 2 * NB]
        ssem = rest[3 * NI + 2 * NB:3 * NI + 3 * NB]
        acc = rest[3 * NI + 3 * NB]
        c = lax.axis_index("c")
        s = lax.axis_index("s")
        ct = jnp.minimum(jnp.maximum(cks - s * cpt, 0), cpt)
        base = (c * cks + s * cpt) * CK

        def istart(q, j):
            off = base + j * CK
            pltpu.async_copy(ei_hbm.at[pl.ds(off, CK)], srci[q], isem[q])
            pltpu.async_copy(ei_hbm.at[pl.ds(E + off, CK)], dsti[q], isem[q])

        def iwait(q):
            pltpu.make_async_copy(ei_hbm.at[pl.ds(0, CK)], srci[q],
                                  isem[q]).wait()
            pltpu.make_async_copy(ei_hbm.at[pl.ds(0, CK)], dsti[q],
                                  isem[q]).wait()

        def gstart(b, q):
            pltpu.async_copy(xws_hbm.at[srci[q]], rows[b], gsem[b])

        def gwait(b):
            pltpu.make_async_copy(xws_hbm.at[pl.ds(0, CK)], rows[b],
                                  gsem[b]).wait()

        def sstart(b, q):
            pltpu.async_copy(rows[b], acc.at[dsti[q]], ssem[b], add=True)

        def swait(b, q):
            pltpu.make_async_copy(rows[b], acc.at[dsti[q]], ssem[b]).wait()

        sr = -(-N // NS // 8) * 8
        lastr = N - (NS - 1) * sr

        @pl.when(jnp.logical_and(c == 0, s < NS - 1))
        def _():
            pltpu.sync_copy(xws_hbm.at[pl.ds(s * sr, sr)],
                            acc.at[pl.ds(s * sr, sr)])

        @pl.when(jnp.logical_and(c == 0, s == NS - 1))
        def _():
            pltpu.sync_copy(xws_hbm.at[pl.ds((NS - 1) * sr, lastr)],
                            acc.at[pl.ds((NS - 1) * sr, lastr)])
            pltpu.sync_copy(zeros_hbm.at[pl.ds(0, npad - N)],
                            acc.at[pl.ds(N, npad - N)])

        @pl.when(c == 1)
        def _():
            pltpu.sync_copy(zeros_hbm, acc.at[pl.ds(s * rpt, rpt)])
        plsc.subcore_barrier()

        for q in range(NI):
            @pl.when(q < ct)
            def _(q=q):
                istart(q, q)
        for b in range(NB):
            @pl.when(b < ct)
            def _(b=b):
                iwait(b)
                gstart(b, b)

        def body(k, carry):
            for u in range(NI):
                j = k * NI + u
                b = u % NB
                q = u

                @pl.when(j < ct)
                def _(j=j, b=b, q=q):
                    gwait(b)
                    sstart(b, q)

                @pl.when(j + NB < ct)
                def _(j=j, b=b, q=q):
                    swait(b, q)
                    iwait((q + NB) % NI)
                    gstart(b, (q + NB) % NI)

                @pl.when(jnp.logical_and(j + NB >= ct, j < ct))
                def _(j=j, b=b, q=q):
                    swait(b, q)

                @pl.when(j + NI < ct)
                def _(j=j, q=q):
                    istart(q, j + NI)


            return carry

        lax.fori_loop(0, cpt // NI, body, 0)
        plsc.subcore_barrier()
        pltpu.sync_copy(acc.at[pl.ds(s * rpt, rpt)],
                        out_hbm.at[c, pl.ds(s * rpt, rpt)])

    return agg


def _pre_call(x, gcn_weight, cnt, R):
    N, F = x.shape
    H = gcn_weight.shape[1]

    def body(x_ref, w_ref, cnt_ref, xws_ref):
        deg = cnt_ref[0, :, 0:1] + cnt_ref[1, :, 0:1] + 1.0
        dinv = lax.rsqrt(deg)
        xw = jnp.dot(x_ref[...], w_ref[...],
                     preferred_element_type=jnp.float32)
        xws_ref[...] = xw * dinv

    grid = (N // R,)
    return pl.pallas_call(
        body,
        grid=grid,
        in_specs=[
            pl.BlockSpec((R, F), lambda i: (i, 0)),
            pl.BlockSpec((F, H), lambda i: (0, 0)),
            pl.BlockSpec((NC, R, 1), lambda i: (0, i, 0)),
        ],
        out_specs=pl.BlockSpec((R, H), lambda i: (i, 0)),
        out_shape=jax.ShapeDtypeStruct((N, H), jnp.float32),
    )(x, gcn_weight, cnt)


def _post_call(aggp, cnt, gcn_bias, whh_t, bhh, wih_t, bih, h0, R):
    N, H = h0.shape
    H3 = 3 * H

    def body(agg_ref, cnt_ref, b_ref, whht_ref, bhh_ref,
             wiht_ref, bih_ref, h0_ref, out_ref):
        deg = cnt_ref[0, :, 0:1] + cnt_ref[1, :, 0:1] + 1.0
        dinv = lax.rsqrt(deg)
        hs = (agg_ref[0] + agg_ref[1]) * dinv + b_ref[...]
        hs = jnp.maximum(hs, 0.0)
        h0v = h0_ref[...]
        gi = jnp.dot(hs, wiht_ref[...],
                     preferred_element_type=jnp.float32) + bih_ref[...]
        gh = jnp.dot(h0v, whht_ref[...],
                     preferred_element_type=jnp.float32) + bhh_ref[...]
        r = jax.nn.sigmoid(gi[:, :H] + gh[:, :H])
        z = jax.nn.sigmoid(gi[:, H:2 * H] + gh[:, H:2 * H])
        n = jnp.tanh(gi[:, 2 * H:] + r * gh[:, 2 * H:])
        out_ref[...] = (1.0 - z) * n + z * h0v

    grid = (N // R,)
    return pl.pallas_call(
        body,
        grid=grid,
        in_specs=[
            pl.BlockSpec((NC, R, H), lambda i: (0, i, 0)),
            pl.BlockSpec((NC, R, 1), lambda i: (0, i, 0)),
            pl.BlockSpec((1, H), lambda i: (0, 0)),
            pl.BlockSpec((H, H3), lambda i: (0, 0)),
            pl.BlockSpec((1, H3), lambda i: (0, 0)),
            pl.BlockSpec((H, H3), lambda i: (0, 0)),
            pl.BlockSpec((1, H3), lambda i: (0, 0)),
            pl.BlockSpec((R, H), lambda i: (i, 0)),
        ],
        out_specs=pl.BlockSpec((R, H), lambda i: (i, 0)),
        out_shape=jax.ShapeDtypeStruct((N, H), jnp.float32),
    )(aggp, cnt, gcn_bias, whh_t, bhh, wih_t, bih, h0)


def kernel(x, edge_index, hidden_state, gcn_weight, gcn_bias,
           w_ih, w_hh, b_ih, b_hh):
    N, F = x.shape
    E = edge_index.shape[1]
    H = gcn_weight.shape[1]
    ei = edge_index.reshape(2 * E)
    h0 = hidden_state[0]
    whh_t = w_hh.T
    wih_t = w_ih.T
    bhh = b_hh.reshape(1, -1)
    bih = b_ih.reshape(1, -1)
    bias = gcn_bias.reshape(1, -1)

    npad = -(-N // (8 * NS)) * (8 * NS)
    rpt = npad // NS
    zeros_agg = jnp.zeros((rpt, H), jnp.float32)
    zeros_deg = jnp.zeros((rpt,), jnp.float32)
    ones_deg = jnp.ones((CK,), jnp.float32)

    cnt = _deg_kernel(E, N)(ei, zeros_deg, ones_deg).reshape(NC, npad, 1)
    xws = _pre_call(x, gcn_weight, cnt, R=1000)
    aggp = _agg_kernel(E, N, H)(xws, ei, zeros_agg)
    h_new = _post_call(aggp, cnt, bias, whh_t, bhh, wih_t, bih, h0, R=1000)
    return h_new, h_new[None]


# speedup vs baseline: 41.1326x; 1.0264x over previous
import functools

import jax
import jax.numpy as jnp
from jax import lax
from jax.experimental import pallas as pl
from jax.experimental.pallas import tpu as pltpu
from jax.experimental.pallas import tpu_sc as plsc

NC = 2
NS = 16
CK = 64
NB = 4


def _tile_plan(E):
    epc = E // NC
    cks = epc // CK
    cpt = -(-cks // NS)
    cpt = -(-cpt // (2 * NB)) * (2 * NB)
    rows_pad = cpt * NS
    return epc, cks, cpt, rows_pad


def _pad_edges(idx, E):
    _, cks, _, rows_pad = _tile_plan(E)
    r = idx.reshape(NC, cks, CK)
    r = jnp.pad(r, ((0, 0), (0, rows_pad - cks), (0, 0)))
    return r.reshape(NC * rows_pad, CK)


def _deg_kernel(E, N):
    _, cks, cpt, rows_pad = _tile_plan(E)
    npad = -(-N // (8 * NS)) * (8 * NS)
    rpt = npad // NS
    mesh = plsc.VectorSubcoreMesh(core_axis_name="c", subcore_axis_name="s")
    NI = 8
    lag = 4

    scratch = [pltpu.VMEM((CK,), jnp.int32) for _ in range(NI)]
    scratch += [
        pltpu.VMEM((CK,), jnp.float32),
        pltpu.VMEM((rpt,), jnp.float32),
        pltpu.VMEM_SHARED((npad,), jnp.float32),
    ]
    scratch += [pltpu.SemaphoreType.DMA for _ in range(NI + 1)]

    @functools.partial(
        pl.kernel,
        mesh=mesh,
        out_type=jax.ShapeDtypeStruct((NC * npad,), jnp.float32),
        scratch_types=scratch,
    )
    def deg(ei_hbm, zeros_hbm, ones_hbm, out_hbm, *rest):
        dsti = rest[:NI]
        onesb = rest[NI]
        zbuf = rest[NI + 1]
        acc = rest[NI + 2]
        isem = rest[NI + 3:2 * NI + 3]
        ssem = rest[2 * NI + 3]
        c = lax.axis_index("c")
        s = lax.axis_index("s")
        ct = jnp.minimum(jnp.maximum(cks - s * cpt, 0), cpt)
        base = E + (c * cks + s * cpt) * CK

        def istart(q, j):
            pltpu.async_copy(ei_hbm.at[pl.ds(base + j * CK, CK)],
                             dsti[q], isem[q])

        def iwait(q):
            pltpu.make_async_copy(ei_hbm.at[pl.ds(0, CK)], dsti[q],
                                  isem[q]).wait()

        pltpu.sync_copy(ones_hbm, onesb)
        pltpu.sync_copy(zeros_hbm, zbuf)
        pltpu.sync_copy(zbuf, acc.at[pl.ds(s * rpt, rpt)])
        plsc.subcore_barrier()

        for q in range(NI):
            @pl.when(q < ct)
            def _(q=q):
                istart(q, q)

        def body(k, carry):
            for q in range(NI):
                j = k * NI + q

                @pl.when(j < ct)
                def _(j=j, q=q):
                    iwait(q)
                    pltpu.async_copy(onesb, acc.at[dsti[q]], ssem, add=True)

                @pl.when(jnp.logical_and(j >= lag, j - lag < ct))
                def _():
                    pltpu.make_async_copy(onesb, acc.at[dsti[0]],
                                          ssem).wait()

                @pl.when(j + NI < ct)
                def _(j=j, q=q):
                    istart(q, j + NI)

            return carry

        lax.fori_loop(0, cpt // NI, body, 0)

        def drain(j, carry):
            @pl.when(j + (cpt - lag) < ct)
            def _():
                pltpu.make_async_copy(onesb, acc.at[dsti[0]], ssem).wait()

            return carry

        lax.fori_loop(0, lag, drain, 0)
        plsc.subcore_barrier()
        pltpu.sync_copy(acc.at[pl.ds(s * rpt, rpt)], zbuf)
        pltpu.sync_copy(zbuf, out_hbm.at[pl.ds(c * npad + s * rpt, rpt)])

    return deg


def _agg_kernel(E, N, H):
    _, cks, cpt, rows_pad = _tile_plan(E)
    npad = -(-N // (8 * NS)) * (8 * NS)
    rpt = npad // NS
    mesh = plsc.VectorSubcoreMesh(core_axis_name="c", subcore_axis_name="s")
    NI = 2 * NB

    scratch = []
    scratch += [pltpu.VMEM((CK,), jnp.int32) for _ in range(NI)]
    scratch += [pltpu.VMEM((CK,), jnp.int32) for _ in range(NI)]
    scratch += [pltpu.VMEM((CK, H), jnp.float32) for _ in range(NB)]
    scratch += [pltpu.SemaphoreType.DMA for _ in range(NI + 2 * NB)]
    scratch += [pltpu.VMEM_SHARED((npad, H), jnp.float32)]

    @functools.partial(
        pl.kernel,
        mesh=mesh,
        out_type=jax.ShapeDtypeStruct((NC, npad, H), jnp.float32),
        scratch_types=scratch,
    )
    def agg(xws_hbm, ei_hbm, zeros_hbm, out_hbm, *rest):
        srci = rest[:NI]
        dsti = rest[NI:2 * NI]
        rows = rest[2 * NI:2 * NI + NB]
        isem = rest[2 * NI + NB:3 * NI + NB]
        gsem = rest[3 * NI + NB:3 * NI + 2 * NB]
        ssem = rest[3 * NI + 2 * NB:3 * NI + 3 * NB]
        acc = rest[3 * NI + 3 * NB]
        c = lax.axis_index("c")
        s = lax.axis_index("s")
        ct = jnp.minimum(jnp.maximum(cks - s * cpt, 0), cpt)
        base = (c * cks + s * cpt) * CK

        def istart(q, j):
            off = base + j * CK
            pltpu.async_copy(ei_hbm.at[pl.ds(off, CK)], srci[q], isem[q])
            pltpu.async_copy(ei_hbm.at[pl.ds(E + off, CK)], dsti[q], isem[q])

        def iwait(q):
            pltpu.make_async_copy(ei_hbm.at[pl.ds(0, CK)], srci[q],
                                  isem[q]).wait()
            pltpu.make_async_copy(ei_hbm.at[pl.ds(0, CK)], dsti[q],
                                  isem[q]).wait()

        def gstart(b, q):
            pltpu.async_copy(xws_hbm.at[srci[q]], rows[b], gsem[b])

        def gwait(b):
            pltpu.make_async_copy(xws_hbm.at[pl.ds(0, CK)], rows[b],
                                  gsem[b]).wait()

        def sstart(b, q):
            pltpu.async_copy(rows[b], acc.at[dsti[q]], ssem[b], add=True)

        def swait(b, q):
            pltpu.make_async_copy(rows[b], acc.at[dsti[q]], ssem[b]).wait()

        sr = -(-N // NS // 8) * 8
        lastr = N - (NS - 1) * sr

        @pl.when(jnp.logical_and(c == 0, s < NS - 1))
        def _():
            pltpu.sync_copy(xws_hbm.at[pl.ds(s * sr, sr)],
                            acc.at[pl.ds(s * sr, sr)])

        @pl.when(jnp.logical_and(c == 0, s == NS - 1))
        def _():
            pltpu.sync_copy(xws_hbm.at[pl.ds((NS - 1) * sr, lastr)],
                            acc.at[pl.ds((NS - 1) * sr, lastr)])
            pltpu.sync_copy(zeros_hbm.at[pl.ds(0, npad - N)],
                            acc.at[pl.ds(N, npad - N)])

        @pl.when(c == 1)
        def _():
            pltpu.sync_copy(zeros_hbm, acc.at[pl.ds(s * rpt, rpt)])
        plsc.subcore_barrier()

        for q in range(NI):
            @pl.when(q < ct)
            def _(q=q):
                istart(q, q)
        for b in range(NB):
            @pl.when(b < ct)
            def _(b=b):
                iwait(b)
                gstart(b, b)

        def body(k, carry):
            for u in range(NI):
                j = k * NI + u
                b = u % NB
                q = u

                @pl.when(j < ct)
                def _(j=j, b=b, q=q):
                    gwait(b)
                    sstart(b, q)

                @pl.when(j + NB < ct)
                def _(j=j, b=b, q=q):
                    swait(b, q)
                    iwait((q + NB) % NI)
                    gstart(b, (q + NB) % NI)

                @pl.when(jnp.logical_and(j + NB >= ct, j < ct))
                def _(j=j, b=b, q=q):
                    swait(b, q)

                @pl.when(j + NI < ct)
                def _(j=j, q=q):
                    istart(q, j + NI)


            return carry

        lax.fori_loop(0, cpt // NI, body, 0)
        plsc.subcore_barrier()
        pltpu.sync_copy(acc.at[pl.ds(s * rpt, rpt)],
                        out_hbm.at[c, pl.ds(s * rpt, rpt)])

    return agg


def _pre_call(x, gcn_weight, cnt, R):
    N, F = x.shape
    H = gcn_weight.shape[1]

    def body(x_ref, w_ref, cnt_ref, xws_ref):
        deg = cnt_ref[0, :, 0:1] + cnt_ref[1, :, 0:1] + 1.0
        dinv = lax.rsqrt(deg)
        xw = jnp.dot(x_ref[...], w_ref[...],
                     preferred_element_type=jnp.float32)
        xws_ref[...] = xw * dinv

    grid = (N // R,)
    return pl.pallas_call(
        body,
        grid=grid,
        in_specs=[
            pl.BlockSpec((R, F), lambda i: (i, 0)),
            pl.BlockSpec((F, H), lambda i: (0, 0)),
            pl.BlockSpec((NC, R, 1), lambda i: (0, i, 0)),
        ],
        out_specs=pl.BlockSpec((R, H), lambda i: (i, 0)),
        out_shape=jax.ShapeDtypeStruct((N, H), jnp.float32),
    )(x, gcn_weight, cnt)


def _post_call(aggp, cnt, gcn_bias, whh_t, bhh, wih_t, bih, h0, R):
    N, H = h0.shape
    H3 = 3 * H

    def body(agg_ref, cnt_ref, b_ref, whht_ref, bhh_ref,
             wiht_ref, bih_ref, h0_ref, out_ref):
        deg = cnt_ref[0, :, 0:1] + cnt_ref[1, :, 0:1] + 1.0
        dinv = lax.rsqrt(deg)
        hs = (agg_ref[0] + agg_ref[1]) * dinv + b_ref[...]
        hs = jnp.maximum(hs, 0.0)
        h0v = h0_ref[...]
        gi = jnp.dot(hs, wiht_ref[...],
                     preferred_element_type=jnp.float32) + bih_ref[...]
        gh = jnp.dot(h0v, whht_ref[...],
                     preferred_element_type=jnp.float32) + bhh_ref[...]
        r = jax.nn.sigmoid(gi[:, :H] + gh[:, :H])
        z = jax.nn.sigmoid(gi[:, H:2 * H] + gh[:, H:2 * H])
        n = jnp.tanh(gi[:, 2 * H:] + r * gh[:, 2 * H:])
        out_ref[...] = (1.0 - z) * n + z * h0v

    grid = (N // R,)
    return pl.pallas_call(
        body,
        grid=grid,
        in_specs=[
            pl.BlockSpec((NC, R, H), lambda i: (0, i, 0)),
            pl.BlockSpec((NC, R, 1), lambda i: (0, i, 0)),
            pl.BlockSpec((1, H), lambda i: (0, 0)),
            pl.BlockSpec((H, H3), lambda i: (0, 0)),
            pl.BlockSpec((1, H3), lambda i: (0, 0)),
            pl.BlockSpec((H, H3), lambda i: (0, 0)),
            pl.BlockSpec((1, H3), lambda i: (0, 0)),
            pl.BlockSpec((R, H), lambda i: (i, 0)),
        ],
        out_specs=pl.BlockSpec((R, H), lambda i: (i, 0)),
        out_shape=jax.ShapeDtypeStruct((N, H), jnp.float32),
    )(aggp, cnt, gcn_bias, whh_t, bhh, wih_t, bih, h0)


def kernel(x, edge_index, hidden_state, gcn_weight, gcn_bias,
           w_ih, w_hh, b_ih, b_hh):
    N, F = x.shape
    E = edge_index.shape[1]
    H = gcn_weight.shape[1]
    ei = edge_index.reshape(2 * E)
    h0 = hidden_state[0]
    whh_t = w_hh.T
    wih_t = w_ih.T
    bhh = b_hh.reshape(1, -1)
    bih = b_ih.reshape(1, -1)
    bias = gcn_bias.reshape(1, -1)

    npad = -(-N // (8 * NS)) * (8 * NS)
    rpt = npad // NS
    zeros_agg = jnp.zeros((rpt, H), jnp.float32)
    zeros_deg = jnp.zeros((rpt,), jnp.float32)
    ones_deg = jnp.ones((CK,), jnp.float32)

    cnt = _deg_kernel(E, N)(ei, zeros_deg, ones_deg).reshape(NC, npad, 1)
    xws = _pre_call(x, gcn_weight, cnt, R=2000)
    aggp = _agg_kernel(E, N, H)(xws, ei, zeros_agg)
    h_new = _post_call(aggp, cnt, bias, whh_t, bhh, wih_t, bih, h0, R=2000)
    return h_new, h_new[None]
